# baseline (speedup 1.0000x reference)
import jax
import jax.numpy as jnp
from jax import lax
from jax.experimental import pallas as pl
from jax.experimental.pallas import tpu as pltpu

B, T, E, S, MAX_W = 16, 2048, 512, 256, 16
HID = 150

_F32 = jnp.float32
_BF16 = jnp.bfloat16

RPB = 2
NB = B // RPB
M = RPB * T


def _dot(a, b):
    return jnp.dot(a, b, preferred_element_type=_F32)


def _mention_kernel(emb_ref, st_ref, wd_ref,
                    aW1_ref, ab1_ref, aW2_ref, ab2_ref, aW3_ref, ab3_ref,
                    sW1_ref, sb1_ref, sW2_ref, sb2_ref, sW3_ref, sb3_ref,
                    se_ref, sc_ref, embs_ref, atts_ref):
    g = pl.program_id(0)
    slot_w = lax.rem(g, 2) * M
    slot_r = lax.rem(g + 1, 2) * M

    aW1 = aW1_ref[...].astype(_BF16)
    aW2 = aW2_ref[...].astype(_BF16)
    aW3 = aW3_ref[...].astype(_BF16)
    sW1 = sW1_ref[...].astype(_BF16)
    sW2 = sW2_ref[...].astype(_BF16)
    sW3 = sW3_ref[...].astype(_BF16)
    tt = lax.broadcasted_iota(jnp.int32, (T, S), 0)
    dn = (((0,), (0,)), ((), ()))
    dn2 = (((0,), (1,)), ((), ()))

    emb_bf = emb_ref[...].astype(_BF16).reshape(M, E)
    h = jnp.maximum(_dot(emb_bf, aW1) + ab1_ref[...], 0.0).astype(_BF16)
    h = jnp.maximum(_dot(h, aW2) + ab2_ref[...], 0.0).astype(_BF16)
    att = (_dot(h, aW3) + ab3_ref[0]).astype(_BF16)
    embs_ref[pl.ds(slot_w, M), :] = emb_bf
    atts_ref[pl.ds(slot_w, M), :] = att

    pemb = embs_ref[pl.ds(slot_r, M), :]
    patt = atts_ref[pl.ds(slot_r, M), :]
    results = []
    for i in range(RPB):
        starts = st_ref[i]
        ends = starts + wd_ref[i]
        in_span = ((tt >= starts) & (tt <= ends)).astype(_BF16)
        oh_start = (tt == starts).astype(_BF16)
        oh_end = (tt == ends).astype(_BF16)

        att_i = patt[i * T:(i + 1) * T]
        emb_i = pemb[i * T:(i + 1) * T]
        big = jnp.concatenate([oh_start, oh_end, in_span * att_i],
                              axis=1)
        res = lax.dot_general(big, emb_i, dn,
                              preferred_element_type=_F32)
        start_emb = res[0:S]
        end_emb = res[S:2 * S]
        weighted = res[2 * S:3 * S]

        se_ref[i, :, 0:E] = start_emb
        se_ref[i, :, E:2 * E] = end_emb
        se_ref[i, :, 2 * E:3 * E] = weighted
        results.append((start_emb, end_emb, weighted))

    x = jnp.concatenate(
        [jnp.concatenate(r, axis=1) for r in results], axis=0).astype(_BF16)
    hs = jnp.maximum(_dot(x, sW1) + sb1_ref[...], 0.0).astype(_BF16)
    hs = jnp.maximum(_dot(hs, sW2) + sb2_ref[...], 0.0).astype(_BF16)
    sc = (lax.dot_general(sW3, hs, dn2, preferred_element_type=_F32)
          + sb3_ref[0])
    for i in range(RPB):
        sc_ref[i] = sc[:, i * S:(i + 1) * S]


def kernel(batch_embeds, span_starts, span_widths, attn_params, score_params):
    aW1, ab1, aW2, ab2, aW3, ab3 = attn_params
    sW1, sb1, sW2, sb2, sW3, sb3 = score_params

    st3 = span_starts.reshape(B, 1, S).astype(jnp.int32)
    wd3 = span_widths.reshape(B, 1, S).astype(jnp.int32)

    def _w(shape):
        return pl.BlockSpec(shape, lambda g: (0,) * len(shape))

    def _s():
        return pl.BlockSpec(memory_space=pltpu.SMEM)

    def _cur(g):
        return (jnp.minimum(g, NB - 1), 0, 0)

    def _prev(g):
        return (jnp.maximum(g - 1, 0), 0, 0)

    in_specs = [
            pl.BlockSpec((RPB, T, E), _cur),
            pl.BlockSpec((RPB, 1, S), _prev),
            pl.BlockSpec((RPB, 1, S), _prev),
            _w((E, HID)), _w((1, HID)), _w((HID, HID)), _w((1, HID)),
            _w((HID, 1)), _s(),
            _w((3 * E, HID)), _w((1, HID)), _w((HID, HID)), _w((1, HID)),
            _w((HID, 1)), _s(),
    ]
    out_specs = [
        pl.BlockSpec((RPB, S, 3 * E), _prev),
        pl.BlockSpec((RPB, 1, S), _prev),
    ]

    span_embeds, scores = pl.pallas_call(
        _mention_kernel,
        grid=(NB + 1,),
        in_specs=in_specs,
        out_specs=out_specs,
        out_shape=[
            jax.ShapeDtypeStruct((B, S, 3 * E), _F32),
            jax.ShapeDtypeStruct((B, 1, S), _F32),
        ],
        scratch_shapes=[
            pltpu.VMEM((2 * M, E), _BF16),
            pltpu.VMEM((2 * M, 1), _BF16),
        ],
        compiler_params=pltpu.CompilerParams(
            dimension_semantics=("arbitrary",),
            vmem_limit_bytes=100 * 1024 * 1024,
        ),
    )(batch_embeds, st3, wd3,
      aW1, ab1.reshape(1, HID), aW2, ab2.reshape(1, HID), aW3, ab3,
      sW1, sb1.reshape(1, HID), sW2, sb2.reshape(1, HID), sW3, sb3)

    return span_embeds, scores.reshape(B, S, 1)


# speedup vs baseline: 1.1270x; 1.1270x over previous
import jax
import jax.numpy as jnp
from jax import lax
from jax.experimental import pallas as pl
from jax.experimental.pallas import tpu as pltpu

B, T, E, S, MAX_W = 16, 2048, 512, 256, 16
HID = 150

_F32 = jnp.float32
_BF16 = jnp.bfloat16


def _dot(a, b):
    return jnp.dot(a, b, preferred_element_type=_F32)


RPB = 2


def _mention_kernel(emb_ref, st_ref, wd_ref,
                    aW1_ref, ab1_ref, aW2_ref, ab2_ref, aW3_ref, ab3_ref,
                    sW1_ref, sb1_ref, sW2_ref, sb2_ref, sW3_ref, sb3_ref,
                    se_ref, sc_ref):
    aW1 = aW1_ref[...].astype(_BF16)
    aW2 = aW2_ref[...].astype(_BF16)
    aW3 = aW3_ref[...].astype(_BF16)
    sW1 = sW1_ref[...].astype(_BF16)
    sW2 = sW2_ref[...].astype(_BF16)
    sW3 = sW3_ref[...].astype(_BF16)
    tt = lax.broadcasted_iota(jnp.int32, (T, S), 0)
    dn = (((0,), (0,)), ((), ()))
    dn2 = (((0,), (1,)), ((), ()))

    emb_bf = emb_ref[...].astype(_BF16).reshape(RPB * T, E)
    h = jnp.maximum(_dot(emb_bf, aW1) + ab1_ref[...], 0.0).astype(_BF16)
    h = jnp.maximum(_dot(h, aW2) + ab2_ref[...], 0.0).astype(_BF16)
    att = (_dot(h, aW3) + ab3_ref[0]).astype(_BF16)

    results = []
    for i in range(RPB):
        starts = st_ref[i]
        ends = starts + wd_ref[i]
        in_span = ((tt >= starts) & (tt <= ends)).astype(_BF16)
        oh_start = (tt == starts).astype(_BF16)
        oh_end = (tt == ends).astype(_BF16)

        att_i = att[i * T:(i + 1) * T]
        emb_i = emb_bf[i * T:(i + 1) * T]
        big = jnp.concatenate([oh_start, oh_end, in_span * att_i],
                              axis=1)
        res = lax.dot_general(big, emb_i, dn,
                              preferred_element_type=_F32)
        start_emb = res[0:S]
        end_emb = res[S:2 * S]
        weighted = res[2 * S:3 * S]

        se_ref[i, :, 0:E] = start_emb
        se_ref[i, :, E:2 * E] = end_emb
        se_ref[i, :, 2 * E:3 * E] = weighted
        results.append((start_emb, end_emb, weighted))

    x = jnp.concatenate(
        [jnp.concatenate(r, axis=1) for r in results], axis=0).astype(_BF16)
    hs = jnp.maximum(_dot(x, sW1) + sb1_ref[...], 0.0).astype(_BF16)
    hs = jnp.maximum(_dot(hs, sW2) + sb2_ref[...], 0.0).astype(_BF16)
    sc = (lax.dot_general(sW3, hs, dn2, preferred_element_type=_F32)
          + sb3_ref[0])
    for i in range(RPB):
        sc_ref[i] = sc[:, i * S:(i + 1) * S]


def kernel(batch_embeds, span_starts, span_widths, attn_params, score_params):
    aW1, ab1, aW2, ab2, aW3, ab3 = attn_params
    sW1, sb1, sW2, sb2, sW3, sb3 = score_params

    st3 = span_starts.reshape(B, 1, S).astype(jnp.int32)
    wd3 = span_widths.reshape(B, 1, S).astype(jnp.int32)

    def _w(shape):
        return pl.BlockSpec(shape, lambda b: (0,) * len(shape))

    def _s():
        return pl.BlockSpec(memory_space=pltpu.SMEM)

    grid_spec = pl.GridSpec(
        grid=(B // RPB,),
        in_specs=[
            pl.BlockSpec((RPB, T, E), lambda b: (b, 0, 0)),
            pl.BlockSpec((RPB, 1, S), lambda b: (b, 0, 0)),
            pl.BlockSpec((RPB, 1, S), lambda b: (b, 0, 0)),
            _w((E, HID)), _w((1, HID)), _w((HID, HID)), _w((1, HID)),
            _w((HID, 1)), _s(),
            _w((3 * E, HID)), _w((1, HID)), _w((HID, HID)), _w((1, HID)),
            _w((HID, 1)), _s(),
        ],
        out_specs=[
            pl.BlockSpec((RPB, S, 3 * E), lambda b: (b, 0, 0)),
            pl.BlockSpec((RPB, 1, S), lambda b: (b, 0, 0)),
        ],
    )

    span_embeds, scores = pl.pallas_call(
        _mention_kernel,
        grid_spec=grid_spec,
        out_shape=[
            jax.ShapeDtypeStruct((B, S, 3 * E), _F32),
            jax.ShapeDtypeStruct((B, 1, S), _F32),
        ],
        compiler_params=pltpu.CompilerParams(
            dimension_semantics=("parallel",),
            vmem_limit_bytes=100 * 1024 * 1024,
        ),
    )(batch_embeds, st3, wd3,
      aW1, ab1.reshape(1, HID), aW2, ab2.reshape(1, HID), aW3, ab3,
      sW1, sb1.reshape(1, HID), sW2, sb2.reshape(1, HID), sW3, sb3)

    return span_embeds, scores.reshape(B, S, 1)
